# baseline (speedup 1.0000x reference)
import functools

import jax
import jax.numpy as jnp
from jax import lax
from jax.experimental import pallas as pl
from jax.experimental.pallas import tpu as pltpu
from jax.experimental.pallas import tpu_sc as plsc

HIDDEN = 2048
EXPERTS = 16
MIN_K = 2
N_TOKENS = 16384
N_WORKERS = 32
ROWS_PER_W = N_TOKENS // N_WORKERS
GROUPS = ROWS_PER_W // 16


def _logits_body(h_ref, w_ref, logits_ref):
    h = h_ref[...]
    w = w_ref[...]

    colnorm = jnp.sqrt(jnp.sum(w * w, axis=0, keepdims=True))
    wn = w / jnp.maximum(colnorm, 1e-12)

    rownorm = jnp.sqrt(jnp.sum(h * h, axis=1, keepdims=True))
    hn = h / jnp.maximum(rownorm, 1e-12)
    logits_ref[...] = jnp.dot(hn, wn, preferred_element_type=jnp.float32)


@functools.partial(jax.jit, static_argnames=("blk",))
def _logits_tc(flat_h, sim_matrix, blk):
    n = flat_h.shape[0]
    return pl.pallas_call(
        _logits_body,
        grid=(n // blk,),
        in_specs=[
            pl.BlockSpec((blk, HIDDEN), lambda i: (i, 0)),
            pl.BlockSpec((HIDDEN, EXPERTS), lambda i: (0, 0)),
        ],
        out_specs=pl.BlockSpec((blk, EXPERTS), lambda i: (i, 0)),
        out_shape=jax.ShapeDtypeStruct((n, EXPERTS), jnp.float32),
    )(flat_h, sim_matrix)


def _routing_sc_body(logits_hbm, gates_hbm, temp_hbm, out_hbm, lv, sc16):
    wid = lax.axis_index("s") * 2 + lax.axis_index("c")
    base = wid * ROWS_PER_W

    pltpu.sync_copy(gates_hbm, sc16.at[0])
    pltpu.sync_copy(temp_hbm, sc16.at[1])
    pltpu.sync_copy(logits_hbm.at[pl.ds(base, ROWS_PER_W)], lv)

    tv = sc16[1]
    s = 1.0 / (1.0 + jnp.exp(-tv))
    sg = sc16[0] * s
    sc16[2] = sg

    iota = lax.iota(jnp.int32, 16)
    neg_inf = jnp.full((16,), -jnp.inf, jnp.float32)
    ones = jnp.full((16,), 1.0, jnp.float32)
    zeros = jnp.full((16,), 0.0, jnp.float32)

    sg_e = [plsc.load_gather(sc16.at[2], [jnp.full((16,), e, jnp.int32)])
            for e in range(EXPERTS)]

    def group(g, carry):
        rows = g * 16 + iota
        vs = [plsc.load_gather(lv, [rows, jnp.full((16,), e, jnp.int32)])
              for e in range(EXPERTS)]

        mb = []
        cnt = zeros
        for e in range(EXPERTS):
            gated = jnp.maximum(vs[e] * s - sg_e[e], 0.0)
            m = gated > 0.0
            mb.append(m)
            cnt = cnt + jnp.where(m, ones, zeros)
        inactive = cnt == 0.0

        m1 = vs[0]
        for e in range(1, EXPERTS):
            m1 = jnp.maximum(m1, vs[e])
        i1 = jnp.full((16,), EXPERTS, jnp.int32)
        for e in range(EXPERTS - 1, -1, -1):
            i1 = jnp.where(vs[e] == m1, jnp.full((16,), e, jnp.int32), i1)
        m2 = neg_inf
        for e in range(EXPERTS):
            m2 = jnp.maximum(m2, jnp.where(i1 == e, neg_inf, vs[e]))
        i2 = jnp.full((16,), EXPERTS, jnp.int32)
        for e in range(EXPERTS - 1, -1, -1):
            hit = jnp.where(i1 == e, neg_inf, vs[e]) == m2
            i2 = jnp.where(hit, jnp.full((16,), e, jnp.int32), i2)

        for e in range(EXPERTS):
            fb = (i1 == e) | (i2 == e)
            val = jnp.where(inactive, jnp.where(fb, ones, zeros),
                            jnp.where(mb[e], ones, zeros))
            plsc.store_scatter(lv, [rows, jnp.full((16,), e, jnp.int32)], val)
        return carry

    lax.fori_loop(0, GROUPS, group, 0)
    pltpu.sync_copy(lv, out_hbm.at[pl.ds(base, ROWS_PER_W)])


@jax.jit
def _routing_sc(logits, gates16, temp16):
    mesh = plsc.VectorSubcoreMesh(core_axis_name="c", subcore_axis_name="s")
    f = pl.kernel(
        _routing_sc_body,
        mesh=mesh,
        compiler_params=pltpu.CompilerParams(needs_layout_passes=False),
        out_type=jax.ShapeDtypeStruct((N_TOKENS, EXPERTS), jnp.float32),
        scratch_types=[
            pltpu.VMEM((ROWS_PER_W, EXPERTS), jnp.float32),
            pltpu.VMEM((3, 16), jnp.float32),
        ],
    )
    return f(logits, gates16, temp16)


def kernel(hidden_states, sim_matrix, gates, temperature):
    b, t, c = hidden_states.shape
    flat_h = hidden_states.reshape(b * t, c)
    logits = _logits_tc(flat_h, sim_matrix, blk=2048)
    gates16 = gates.reshape(EXPERTS).astype(jnp.float32)
    temp16 = jnp.broadcast_to(temperature.astype(jnp.float32), (16,))
    mask = _routing_sc(logits, gates16, temp16)
    return (mask, logits)


# speedup vs baseline: 1.0035x; 1.0035x over previous
import functools

import jax
import jax.numpy as jnp
from jax import lax
from jax.experimental import pallas as pl
from jax.experimental.pallas import tpu as pltpu
from jax.experimental.pallas import tpu_sc as plsc

HIDDEN = 2048
EXPERTS = 16
MIN_K = 2
N_TOKENS = 16384
N_WORKERS = 32
ROWS_PER_W = N_TOKENS // N_WORKERS
GROUPS = ROWS_PER_W // 16


def _logits_body(h_ref, w_ref, logits_ref):
    h = h_ref[...]
    w = w_ref[...]

    colnorm = jnp.sqrt(jnp.sum(w * w, axis=0, keepdims=True))
    wn = w / jnp.maximum(colnorm, 1e-12)

    rownorm = jnp.sqrt(jnp.sum(h * h, axis=1, keepdims=True))
    hn = h / jnp.maximum(rownorm, 1e-12)
    logits_ref[...] = jnp.dot(hn, wn, preferred_element_type=jnp.float32)


@functools.partial(jax.jit, static_argnames=("blk",))
def _logits_tc(flat_h, sim_matrix, blk):
    n = flat_h.shape[0]
    return pl.pallas_call(
        _logits_body,
        grid=(n // blk,),
        in_specs=[
            pl.BlockSpec((blk, HIDDEN), lambda i: (i, 0)),
            pl.BlockSpec((HIDDEN, EXPERTS), lambda i: (0, 0)),
        ],
        out_specs=pl.BlockSpec((blk, EXPERTS), lambda i: (i, 0)),
        out_shape=jax.ShapeDtypeStruct((n, EXPERTS), jnp.float32),
    )(flat_h, sim_matrix)


def _routing_sc_body(logits_hbm, gates_hbm, temp_hbm, out_hbm, lv, sc16):
    wid = lax.axis_index("s") * 2 + lax.axis_index("c")
    base = wid * ROWS_PER_W

    pltpu.sync_copy(gates_hbm, sc16.at[0])
    pltpu.sync_copy(temp_hbm, sc16.at[1])
    pltpu.sync_copy(logits_hbm.at[pl.ds(base, ROWS_PER_W)], lv)

    tv = sc16[1]
    s = 1.0 / (1.0 + jnp.exp(-tv))
    sg = sc16[0] * s
    sc16[2] = sg

    iota = lax.iota(jnp.int32, 16)
    neg_inf = jnp.full((16,), -jnp.inf, jnp.float32)
    ones = jnp.full((16,), 1.0, jnp.float32)
    zeros = jnp.full((16,), 0.0, jnp.float32)

    sg_e = [plsc.load_gather(sc16.at[2], [jnp.full((16,), e, jnp.int32)])
            for e in range(EXPERTS)]

    @plsc.parallel_loop(0, GROUPS, 1, unroll=4)
    def group(g):
        rows = g * 16 + iota
        vs = [plsc.load_gather(lv, [rows, jnp.full((16,), e, jnp.int32)])
              for e in range(EXPERTS)]

        mb = [vs[e] * s > sg_e[e] for e in range(EXPERTS)]
        anyact = mb[0]
        for e in range(1, EXPERTS):
            anyact = anyact | mb[e]
        inactive = ~anyact
        n_inact = jnp.max(jnp.where(inactive, 1, 0))

        @pl.when(n_inact > 0)
        def _with_fallback():
            p1 = [jnp.maximum(vs[2 * i], vs[2 * i + 1]) for i in range(8)]
            p2 = [jnp.minimum(vs[2 * i], vs[2 * i + 1]) for i in range(8)]
            while len(p1) > 1:
                q1, q2 = [], []
                for i in range(0, len(p1), 2):
                    q1.append(jnp.maximum(p1[i], p1[i + 1]))
                    q2.append(jnp.maximum(jnp.minimum(p1[i], p1[i + 1]),
                                          jnp.maximum(p2[i], p2[i + 1])))
                p1, p2 = q1, q2
            m1, m2 = p1[0], p2[0]
            t = [jnp.where(vs[e] == m1, jnp.full((16,), e, jnp.int32),
                           jnp.full((16,), EXPERTS, jnp.int32))
                 for e in range(EXPERTS)]
            while len(t) > 1:
                t = [jnp.minimum(t[i], t[i + 1]) for i in range(0, len(t), 2)]
            i1 = t[0]
            u = [jnp.where((vs[e] == m2) & (i1 != e),
                           jnp.full((16,), e, jnp.int32),
                           jnp.full((16,), EXPERTS, jnp.int32))
                 for e in range(EXPERTS)]
            while len(u) > 1:
                u = [jnp.minimum(u[i], u[i + 1]) for i in range(0, len(u), 2)]
            i2 = u[0]
            for e in range(EXPERTS):
                fb = (i1 == e) | (i2 == e)
                val = jnp.where(inactive, jnp.where(fb, ones, zeros),
                                jnp.where(mb[e], ones, zeros))
                plsc.store_scatter(lv, [rows, jnp.full((16,), e, jnp.int32)], val)

        @pl.when(n_inact == 0)
        def _no_fallback():
            for e in range(EXPERTS):
                plsc.store_scatter(lv, [rows, jnp.full((16,), e, jnp.int32)],
                                   jnp.where(mb[e], ones, zeros))

    pltpu.sync_copy(lv, out_hbm.at[pl.ds(base, ROWS_PER_W)])


@jax.jit
def _routing_sc(logits, gates16, temp16):
    mesh = plsc.VectorSubcoreMesh(core_axis_name="c", subcore_axis_name="s")
    f = pl.kernel(
        _routing_sc_body,
        mesh=mesh,
        compiler_params=pltpu.CompilerParams(needs_layout_passes=False),
        out_type=jax.ShapeDtypeStruct((N_TOKENS, EXPERTS), jnp.float32),
        scratch_types=[
            pltpu.VMEM((ROWS_PER_W, EXPERTS), jnp.float32),
            pltpu.VMEM((3, 16), jnp.float32),
        ],
    )
    return f(logits, gates16, temp16)


def kernel(hidden_states, sim_matrix, gates, temperature):
    b, t, c = hidden_states.shape
    flat_h = hidden_states.reshape(b * t, c)
    logits = _logits_tc(flat_h, sim_matrix, blk=2048)
    gates16 = gates.reshape(EXPERTS).astype(jnp.float32)
    temp16 = jnp.broadcast_to(temperature.astype(jnp.float32), (16,))
    mask = _routing_sc(logits, gates16, temp16)
    return (mask, logits)


# speedup vs baseline: 1.0224x; 1.0189x over previous
import functools

import jax
import jax.numpy as jnp
from jax import lax
from jax.experimental import pallas as pl
from jax.experimental.pallas import tpu as pltpu
from jax.experimental.pallas import tpu_sc as plsc

HIDDEN = 2048
EXPERTS = 16
MIN_K = 2
N_TOKENS = 16384
N_WORKERS = 32
ROWS_PER_W = N_TOKENS // N_WORKERS
GROUPS = ROWS_PER_W // 16


def _logits_body(h_ref, w_ref, logits_ref):
    h = h_ref[...]
    w = w_ref[...]

    colnorm = jnp.sqrt(jnp.sum(w * w, axis=0, keepdims=True))
    wn = w / jnp.maximum(colnorm, 1e-12)

    rownorm = jnp.sqrt(jnp.sum(h * h, axis=1, keepdims=True))
    hn = h / jnp.maximum(rownorm, 1e-12)
    logits_ref[...] = jnp.dot(hn, wn, preferred_element_type=jnp.float32)


@functools.partial(jax.jit, static_argnames=("blk",))
def _logits_tc(flat_h, sim_matrix, blk):
    n = flat_h.shape[0]
    return pl.pallas_call(
        _logits_body,
        grid=(n // blk,),
        in_specs=[
            pl.BlockSpec((blk, HIDDEN), lambda i: (i, 0)),
            pl.BlockSpec((HIDDEN, EXPERTS), lambda i: (0, 0)),
        ],
        out_specs=pl.BlockSpec((blk, EXPERTS), lambda i: (i, 0)),
        out_shape=jax.ShapeDtypeStruct((n, EXPERTS), jnp.float32),
    )(flat_h, sim_matrix)


def _routing_sc_body(logits_hbm, params_hbm, out_hbm, lv, sc16, sem):
    wid = lax.axis_index("s") * 2 + lax.axis_index("c")
    base = wid * ROWS_PER_W

    cp1 = pltpu.make_async_copy(params_hbm, sc16.at[pl.ds(0, 2)], sem)
    cp2 = pltpu.make_async_copy(logits_hbm.at[pl.ds(base, ROWS_PER_W)], lv, sem)
    cp1.start()
    cp2.start()
    cp1.wait()
    cp2.wait()

    tv = sc16[1]
    s = 1.0 / (1.0 + jnp.exp(-tv))
    sg = sc16[0] * s
    sc16[2] = sg

    iota = lax.iota(jnp.int32, 16)
    neg_inf = jnp.full((16,), -jnp.inf, jnp.float32)
    ones = jnp.full((16,), 1.0, jnp.float32)
    zeros = jnp.full((16,), 0.0, jnp.float32)

    sg_e = [plsc.load_gather(sc16.at[2], [jnp.full((16,), e, jnp.int32)])
            for e in range(EXPERTS)]

    @plsc.parallel_loop(0, GROUPS, 1, unroll=4)
    def group(g):
        rows = g * 16 + iota
        vs = [plsc.load_gather(lv, [rows, jnp.full((16,), e, jnp.int32)])
              for e in range(EXPERTS)]

        mb = [vs[e] * s > sg_e[e] for e in range(EXPERTS)]
        anyact = mb[0]
        for e in range(1, EXPERTS):
            anyact = anyact | mb[e]
        inactive = ~anyact
        n_inact = jnp.max(jnp.where(inactive, 1, 0))

        @pl.when(n_inact > 0)
        def _with_fallback():
            p1 = [jnp.maximum(vs[2 * i], vs[2 * i + 1]) for i in range(8)]
            p2 = [jnp.minimum(vs[2 * i], vs[2 * i + 1]) for i in range(8)]
            while len(p1) > 1:
                q1, q2 = [], []
                for i in range(0, len(p1), 2):
                    q1.append(jnp.maximum(p1[i], p1[i + 1]))
                    q2.append(jnp.maximum(jnp.minimum(p1[i], p1[i + 1]),
                                          jnp.maximum(p2[i], p2[i + 1])))
                p1, p2 = q1, q2
            m1, m2 = p1[0], p2[0]
            t = [jnp.where(vs[e] == m1, jnp.full((16,), e, jnp.int32),
                           jnp.full((16,), EXPERTS, jnp.int32))
                 for e in range(EXPERTS)]
            while len(t) > 1:
                t = [jnp.minimum(t[i], t[i + 1]) for i in range(0, len(t), 2)]
            i1 = t[0]
            u = [jnp.where((vs[e] == m2) & (i1 != e),
                           jnp.full((16,), e, jnp.int32),
                           jnp.full((16,), EXPERTS, jnp.int32))
                 for e in range(EXPERTS)]
            while len(u) > 1:
                u = [jnp.minimum(u[i], u[i + 1]) for i in range(0, len(u), 2)]
            i2 = u[0]
            for e in range(EXPERTS):
                fb = (i1 == e) | (i2 == e)
                val = jnp.where(inactive, jnp.where(fb, ones, zeros),
                                jnp.where(mb[e], ones, zeros))
                plsc.store_scatter(lv, [rows, jnp.full((16,), e, jnp.int32)], val)

        @pl.when(n_inact == 0)
        def _no_fallback():
            for e in range(EXPERTS):
                plsc.store_scatter(lv, [rows, jnp.full((16,), e, jnp.int32)],
                                   jnp.where(mb[e], ones, zeros))

    pltpu.sync_copy(lv, out_hbm.at[pl.ds(base, ROWS_PER_W)])


@jax.jit
def _routing_sc(logits, params):
    mesh = plsc.VectorSubcoreMesh(core_axis_name="c", subcore_axis_name="s")
    f = pl.kernel(
        _routing_sc_body,
        mesh=mesh,
        compiler_params=pltpu.CompilerParams(needs_layout_passes=False),
        out_type=jax.ShapeDtypeStruct((N_TOKENS, EXPERTS), jnp.float32),
        scratch_types=[
            pltpu.VMEM((ROWS_PER_W, EXPERTS), jnp.float32),
            pltpu.VMEM((3, 16), jnp.float32),
            pltpu.SemaphoreType.DMA,
        ],
    )
    return f(logits, params)


def kernel(hidden_states, sim_matrix, gates, temperature):
    b, t, c = hidden_states.shape
    flat_h = hidden_states.reshape(b * t, c)
    logits = _logits_tc(flat_h, sim_matrix, blk=2048)
    params = jnp.stack([gates.reshape(EXPERTS).astype(jnp.float32),
                        jnp.broadcast_to(temperature.astype(jnp.float32), (16,))])
    mask = _routing_sc(logits, params)
    return (mask, logits)


# speedup vs baseline: 1.4086x; 1.3777x over previous
import functools

import jax
import jax.numpy as jnp
from jax import lax
from jax.experimental import pallas as pl
from jax.experimental.pallas import tpu as pltpu

HIDDEN = 2048
EXPERTS = 16
MIN_K = 2


def _gating_body(h_ref, w_ref, g_ref, t_ref, mask_ref, logits_ref):
    h = h_ref[...]
    w = w_ref[...]

    colnorm = jnp.sqrt(jnp.sum(w * w, axis=0, keepdims=True))
    wn = w / jnp.maximum(colnorm, 1e-12)

    rownorm = jnp.sqrt(jnp.sum(h * h, axis=1, keepdims=True))
    hn = h / jnp.maximum(rownorm, 1e-12)
    logits = jnp.dot(hn, wn, preferred_element_type=jnp.float32)

    s = jax.nn.sigmoid(t_ref[0])
    scaled = logits * s
    sg = g_ref[...] * s
    gated = jnp.maximum(scaled - sg, 0.0)
    mask = (gated > 0.0).astype(jnp.float32)
    inactive = jnp.sum(mask, axis=1, keepdims=True) == 0.0
    n_inact = jnp.sum(inactive.astype(jnp.int32))

    logits_ref[...] = logits

    @pl.when(n_inact > 0)
    def _with_fallback():
        blk = logits.shape[0]
        iota = lax.broadcasted_iota(jnp.int32, (blk, EXPERTS), 1)
        m1 = jnp.max(logits, axis=1, keepdims=True)
        i1 = jnp.min(jnp.where(logits == m1, iota, EXPERTS), axis=1,
                     keepdims=True)
        neg = jnp.float32(-jnp.inf)
        l2 = jnp.where(iota == i1, neg, logits)
        m2 = jnp.max(l2, axis=1, keepdims=True)
        i2 = jnp.min(jnp.where(l2 == m2, iota, EXPERTS), axis=1, keepdims=True)
        fb = ((iota == i1) | (iota == i2)).astype(jnp.float32)
        mask_ref[...] = jnp.where(inactive, fb, mask)

    @pl.when(n_inact == 0)
    def _no_fallback():
        mask_ref[...] = mask


@functools.partial(jax.jit, static_argnames=("blk",))
def _gating(flat_h, sim_matrix, gates, temperature, blk):
    n = flat_h.shape[0]
    grid = (n // blk,)
    return pl.pallas_call(
        _gating_body,
        grid=grid,
        in_specs=[
            pl.BlockSpec((blk, HIDDEN), lambda i: (i, 0)),
            pl.BlockSpec((HIDDEN, EXPERTS), lambda i: (0, 0)),
            pl.BlockSpec((1, EXPERTS), lambda i: (0, 0)),
            pl.BlockSpec(memory_space=pltpu.SMEM),
        ],
        out_specs=[
            pl.BlockSpec((blk, EXPERTS), lambda i: (i, 0)),
            pl.BlockSpec((blk, EXPERTS), lambda i: (i, 0)),
        ],
        out_shape=[
            jax.ShapeDtypeStruct((n, EXPERTS), jnp.float32),
            jax.ShapeDtypeStruct((n, EXPERTS), jnp.float32),
        ],
    )(flat_h, sim_matrix, gates.reshape(1, EXPERTS),
      temperature.reshape(1).astype(jnp.float32))


def kernel(hidden_states, sim_matrix, gates, temperature):
    b, t, c = hidden_states.shape
    flat_h = hidden_states.reshape(b * t, c)
    mask, logits = _gating(flat_h, sim_matrix, gates, temperature, blk=2048)
    return (mask, logits)
